# baseline (speedup 1.0000x reference)
import functools

import jax
import jax.numpy as jnp
import numpy as np
from jax.experimental import pallas as pl
from jax.experimental.pallas import tpu as pltpu
from jax.sharding import Mesh, PartitionSpec as P

_NUM_EMB = 1024
_EMB_DIM = 64
_BLOCK = 2048


def _vq_block_kernel(x_ref, w_ref, out_ref, loss_ref):
    x = x_ref[...]
    w = w_ref[...]
    b = x.shape[0]

    i_norm = jnp.sum(x * x, axis=1, keepdims=True)
    ones_row = jnp.ones((1, _EMB_DIM), dtype=jnp.float32)
    w_norm = jax.lax.dot_general(
        ones_row, w * w, (((1,), (1,)), ((), ())),
        preferred_element_type=jnp.float32)

    neg_mm = jax.lax.dot_general(
        x * -2.0, w, (((1,), (1,)), ((), ())),
        preferred_element_type=jnp.float32)
    dist = (i_norm + w_norm) + neg_mm

    mn = jnp.min(dist, axis=1, keepdims=True)
    iota = jax.lax.broadcasted_iota(jnp.int32, (b, _NUM_EMB), 1)
    cand = jnp.where(dist == mn, iota, jnp.int32(_NUM_EMB))
    idx = jnp.min(cand, axis=1, keepdims=True)

    one_hot = (iota == idx).astype(jnp.float32)
    q = jax.lax.dot_general(
        one_hot, w, (((1,), (0,)), ((), ())),
        preferred_element_type=jnp.float32)

    out_ref[...] = x + (q - x)
    loss_ref[...] = jnp.sum((x - q) ** 2).reshape(1, 1, 1)


def _vq_shard(flat, W):
    n = flat.shape[0]
    grid = n // _BLOCK
    return pl.pallas_call(
        _vq_block_kernel,
        grid=(grid,),
        in_specs=[
            pl.BlockSpec((_BLOCK, _EMB_DIM), lambda i: (i, 0)),
            pl.BlockSpec((_NUM_EMB, _EMB_DIM), lambda i: (0, 0)),
        ],
        out_specs=[
            pl.BlockSpec((_BLOCK, _EMB_DIM), lambda i: (i, 0)),
            pl.BlockSpec((1, 1, 1), lambda i: (i, 0, 0)),
        ],
        out_shape=[
            jax.ShapeDtypeStruct((n, _EMB_DIM), jnp.float32),
            jax.ShapeDtypeStruct((grid, 1, 1), jnp.float32),
        ],
    )(flat, W)


@jax.jit
def kernel(input, W):
    shape = input.shape
    flat = input.reshape(-1, shape[-1])
    n = flat.shape[0]

    devs = jax.devices()
    mesh = Mesh(np.array(devs), ("d",))
    sharded = jax.shard_map(
        _vq_shard, mesh=mesh,
        in_specs=(P("d", None), P(None, None)),
        out_specs=(P("d", None), P("d", None, None)),
        check_vma=False,
    )
    out, loss_parts = sharded(flat, W)

    e_latent_loss = jnp.sum(loss_parts) / jnp.float32(n * _EMB_DIM)
    loss = 0.25 * e_latent_loss
    return (out.reshape(shape), loss.reshape(()))


# speedup vs baseline: 6.2687x; 6.2687x over previous
import jax
import jax.numpy as jnp
from jax.experimental import pallas as pl
from jax.experimental.pallas import tpu as pltpu

_NUM_EMB = 1024
_EMB_DIM = 64
_SEQ = 1024
_ROWS = 2
_BLOCK = _ROWS * _SEQ


def _vq_block_kernel(x_ref, w_ref, out_ref, loss_ref):
    x = x_ref[...].reshape(_BLOCK, _EMB_DIM)
    w = w_ref[...]

    i_norm = jnp.sum(x * x, axis=1, keepdims=True)
    ones_row = jnp.ones((1, _EMB_DIM), dtype=jnp.float32)
    w_norm = jax.lax.dot_general(
        ones_row, w * w, (((1,), (1,)), ((), ())),
        preferred_element_type=jnp.float32)

    neg_mm = jax.lax.dot_general(
        x * -2.0, w, (((1,), (1,)), ((), ())),
        preferred_element_type=jnp.float32)
    dist = (i_norm + w_norm) + neg_mm

    mn = jnp.min(dist, axis=1, keepdims=True)
    iota = jax.lax.broadcasted_iota(jnp.int32, (_BLOCK, _NUM_EMB), 1)
    cand = jnp.where(dist == mn, iota, jnp.int32(_NUM_EMB))
    idx = jnp.min(cand, axis=1, keepdims=True)

    one_hot = (cand == idx).astype(jnp.float32)
    q = jax.lax.dot_general(
        one_hot, w, (((1,), (0,)), ((), ())),
        preferred_element_type=jnp.float32)

    out_ref[...] = (x + (q - x)).reshape(_ROWS, _SEQ, _EMB_DIM)
    loss_ref[...] = jnp.sum((x - q) ** 2).reshape(1, 1, 1)


@jax.jit
def kernel(input, W):
    shape = input.shape
    n = shape[0] * shape[1]
    grid = shape[0] // _ROWS

    out, loss_parts = pl.pallas_call(
        _vq_block_kernel,
        grid=(grid,),
        in_specs=[
            pl.BlockSpec((_ROWS, _SEQ, _EMB_DIM), lambda i: (i, 0, 0)),
            pl.BlockSpec((_NUM_EMB, _EMB_DIM), lambda i: (0, 0)),
        ],
        out_specs=[
            pl.BlockSpec((_ROWS, _SEQ, _EMB_DIM), lambda i: (i, 0, 0)),
            pl.BlockSpec((1, 1, 1), lambda i: (i, 0, 0)),
        ],
        out_shape=[
            jax.ShapeDtypeStruct(shape, jnp.float32),
            jax.ShapeDtypeStruct((grid, 1, 1), jnp.float32),
        ],
        compiler_params=pltpu.CompilerParams(
            dimension_semantics=("parallel",)),
    )(input, W)

    e_latent_loss = jnp.sum(loss_parts) / jnp.float32(n * _EMB_DIM)
    loss = 0.25 * e_latent_loss
    return (out, loss.reshape(()))


# speedup vs baseline: 10.2768x; 1.6394x over previous
import jax
import jax.numpy as jnp
from jax.experimental import pallas as pl
from jax.experimental.pallas import tpu as pltpu

_NUM_EMB = 1024
_EMB_DIM = 64
_SEQ = 1024


def _vq_block_kernel(xt_ref, wt_ref, out_ref, loss_ref):
    xt = xt_ref[...].reshape(_EMB_DIM, _SEQ)
    wt = wt_ref[...]

    i_norm = jnp.sum(xt * xt, axis=0, keepdims=True)
    ones_col = jnp.ones((_EMB_DIM, 1), dtype=jnp.float32)
    w_norm = jax.lax.dot_general(
        wt * wt, ones_col, (((0,), (0,)), ((), ())),
        preferred_element_type=jnp.float32)

    neg_mm = jax.lax.dot_general(
        wt * -2.0, xt, (((0,), (0,)), ((), ())),
        preferred_element_type=jnp.float32)
    dist = (w_norm + i_norm) + neg_mm

    mn = jnp.min(dist, axis=0, keepdims=True)
    iota = jax.lax.broadcasted_iota(jnp.int32, (_NUM_EMB, _SEQ), 0)
    cand = jnp.where(dist == mn, iota, jnp.int32(_NUM_EMB))
    idx = jnp.min(cand, axis=0, keepdims=True)

    one_hot = (cand == idx).astype(jnp.float32)
    q = jax.lax.dot_general(
        wt, one_hot, (((1,), (0,)), ((), ())),
        preferred_element_type=jnp.float32)

    out_ref[...] = (xt + (q - xt)).reshape(1, _EMB_DIM, _SEQ)
    loss_ref[...] = jnp.sum((xt - q) ** 2).reshape(1, 1, 1)


@jax.jit
def kernel(input, W):
    shape = input.shape
    n = shape[0] * shape[1]
    grid = shape[0]

    xt = jnp.swapaxes(input, 1, 2)
    wt = W.T

    out_t, loss_parts = pl.pallas_call(
        _vq_block_kernel,
        grid=(grid,),
        in_specs=[
            pl.BlockSpec((1, _EMB_DIM, _SEQ), lambda i: (i, 0, 0)),
            pl.BlockSpec((_EMB_DIM, _NUM_EMB), lambda i: (0, 0)),
        ],
        out_specs=[
            pl.BlockSpec((1, _EMB_DIM, _SEQ), lambda i: (i, 0, 0)),
            pl.BlockSpec((1, 1, 1), lambda i: (i, 0, 0)),
        ],
        out_shape=[
            jax.ShapeDtypeStruct((shape[0], _EMB_DIM, _SEQ), jnp.float32),
            jax.ShapeDtypeStruct((grid, 1, 1), jnp.float32),
        ],
        compiler_params=pltpu.CompilerParams(
            dimension_semantics=("parallel",)),
    )(xt, wt)

    e_latent_loss = jnp.sum(loss_parts) / jnp.float32(n * _EMB_DIM)
    loss = 0.25 * e_latent_loss
    return (jnp.swapaxes(out_t, 1, 2), loss.reshape(()))


# speedup vs baseline: 11.2062x; 1.0904x over previous
import jax
import jax.numpy as jnp
from jax.experimental import pallas as pl
from jax.experimental.pallas import tpu as pltpu

_NUM_EMB = 1024
_EMB_DIM = 64
_SEQ = 1024


def _vq_block_kernel(xt_ref, wt_ref, icol_ref, out_ref, loss_ref):
    xt = xt_ref[...].reshape(_EMB_DIM, _SEQ)
    wt = wt_ref[...]
    icol = icol_ref[...]

    i_norm = jnp.sum(xt * xt, axis=0, keepdims=True)
    ones_col = jnp.ones((_EMB_DIM, 1), dtype=jnp.float32)
    w_norm = jax.lax.dot_general(
        wt * wt, ones_col, (((0,), (0,)), ((), ())),
        preferred_element_type=jnp.float32)

    neg_mm = jax.lax.dot_general(
        wt * -2.0, xt, (((0,), (0,)), ((), ())),
        preferred_element_type=jnp.float32)

    _C = 256
    mn = None
    idx = None
    for c in range(_NUM_EMB // _C):
        wn_c = jax.lax.slice(w_norm, (c * _C, 0), ((c + 1) * _C, 1))
        nm_c = jax.lax.slice(neg_mm, (c * _C, 0), ((c + 1) * _C, _SEQ))
        d_c = (wn_c + i_norm) + nm_c
        cmn = jnp.min(d_c, axis=0, keepdims=True)
        icol_c = jax.lax.slice(icol, (c * _C, 0), ((c + 1) * _C, 1))
        cand_c = jnp.where(d_c == cmn, icol_c, jnp.float32(_NUM_EMB))
        cidx = jnp.min(cand_c, axis=0, keepdims=True)
        if mn is None:
            mn, idx = cmn, cidx
        else:
            upd = cmn < mn
            idx = jnp.where(upd, cidx, idx)
            mn = jnp.minimum(mn, cmn)

    one_hot = (icol == idx).astype(jnp.float32)
    q = jax.lax.dot_general(
        wt, one_hot, (((1,), (0,)), ((), ())),
        preferred_element_type=jnp.float32)

    out_ref[...] = (xt + (q - xt)).reshape(1, _EMB_DIM, _SEQ)
    loss_ref[...] = jnp.sum((xt - q) ** 2).reshape(1, 1, 1)


@jax.jit
def kernel(input, W):
    shape = input.shape
    n = shape[0] * shape[1]
    grid = shape[0]

    xt = jnp.swapaxes(input, 1, 2)
    wt = W.T
    icol = jax.lax.broadcasted_iota(jnp.float32, (_NUM_EMB, 1), 0)

    out_t, loss_parts = pl.pallas_call(
        _vq_block_kernel,
        grid=(grid,),
        in_specs=[
            pl.BlockSpec((1, _EMB_DIM, _SEQ), lambda i: (i, 0, 0)),
            pl.BlockSpec((_EMB_DIM, _NUM_EMB), lambda i: (0, 0)),
            pl.BlockSpec((_NUM_EMB, 1), lambda i: (0, 0)),
        ],
        out_specs=[
            pl.BlockSpec((1, _EMB_DIM, _SEQ), lambda i: (i, 0, 0)),
            pl.BlockSpec((1, 1, 1), lambda i: (i, 0, 0)),
        ],
        out_shape=[
            jax.ShapeDtypeStruct((shape[0], _EMB_DIM, _SEQ), jnp.float32),
            jax.ShapeDtypeStruct((grid, 1, 1), jnp.float32),
        ],
        compiler_params=pltpu.CompilerParams(
            dimension_semantics=("parallel",)),
    )(xt, wt, icol)

    e_latent_loss = jnp.sum(loss_parts) / jnp.float32(n * _EMB_DIM)
    loss = 0.25 * e_latent_loss
    return (jnp.swapaxes(out_t, 1, 2), loss.reshape(()))


# speedup vs baseline: 13.1289x; 1.1716x over previous
import jax
import jax.numpy as jnp
from jax.experimental import pallas as pl
from jax.experimental.pallas import tpu as pltpu

_NUM_EMB = 1024
_EMB_DIM = 64
_SEQ = 1024


_SLABS = 4


def _vq_slab(xt, wt, icol, w_norm):
    i_norm = jnp.sum(xt * xt, axis=0, keepdims=True)

    neg_mm = jax.lax.dot_general(
        wt * -2.0, xt, (((0,), (0,)), ((), ())),
        preferred_element_type=jnp.float32)

    _C = 256
    mn = None
    idx = None
    for c in range(_NUM_EMB // _C):
        wn_c = jax.lax.slice(w_norm, (c * _C, 0), ((c + 1) * _C, 1))
        nm_c = jax.lax.slice(neg_mm, (c * _C, 0), ((c + 1) * _C, _SEQ))
        d_c = (wn_c + i_norm) + nm_c
        cmn = jnp.min(d_c, axis=0, keepdims=True)
        icol_c = jax.lax.slice(icol, (c * _C, 0), ((c + 1) * _C, 1))
        cand_c = jnp.where(d_c == cmn, icol_c, jnp.float32(_NUM_EMB))
        cidx = jnp.min(cand_c, axis=0, keepdims=True)
        if mn is None:
            mn, idx = cmn, cidx
        else:
            upd = cmn < mn
            idx = jnp.where(upd, cidx, idx)
            mn = jnp.minimum(mn, cmn)

    one_hot = (icol == idx).astype(jnp.float32)
    q = jax.lax.dot_general(
        wt, one_hot, (((1,), (0,)), ((), ())),
        preferred_element_type=jnp.float32)

    return xt + (q - xt), jnp.sum((xt - q) ** 2)


def _vq_block_kernel(xt_ref, wt_ref, icol_ref, out_ref, loss_ref):
    wt = wt_ref[...]
    icol = icol_ref[...]

    ones_col = jnp.ones((_EMB_DIM, 1), dtype=jnp.float32)
    w_norm = jax.lax.dot_general(
        wt * wt, ones_col, (((0,), (0,)), ((), ())),
        preferred_element_type=jnp.float32)

    loss = jnp.zeros((), dtype=jnp.float32)
    for s in range(_SLABS):
        xt = xt_ref[s]
        out, part = _vq_slab(xt, wt, icol, w_norm)
        out_ref[s] = out
        loss = loss + part
    loss_ref[...] = loss.reshape(1, 1, 1)


@jax.jit
def kernel(input, W):
    shape = input.shape
    n = shape[0] * shape[1]
    grid = shape[0] // _SLABS

    xt = jnp.swapaxes(input, 1, 2)
    wt = W.T
    icol = jax.lax.broadcasted_iota(jnp.float32, (_NUM_EMB, 1), 0)

    out_t, loss_parts = pl.pallas_call(
        _vq_block_kernel,
        grid=(grid,),
        in_specs=[
            pl.BlockSpec((_SLABS, _EMB_DIM, _SEQ), lambda i: (i, 0, 0)),
            pl.BlockSpec((_EMB_DIM, _NUM_EMB), lambda i: (0, 0)),
            pl.BlockSpec((_NUM_EMB, 1), lambda i: (0, 0)),
        ],
        out_specs=[
            pl.BlockSpec((_SLABS, _EMB_DIM, _SEQ), lambda i: (i, 0, 0)),
            pl.BlockSpec((1, 1, 1), lambda i: (i, 0, 0)),
        ],
        out_shape=[
            jax.ShapeDtypeStruct((shape[0], _EMB_DIM, _SEQ), jnp.float32),
            jax.ShapeDtypeStruct((grid, 1, 1), jnp.float32),
        ],
        compiler_params=pltpu.CompilerParams(
            dimension_semantics=("parallel",)),
    )(xt, wt, icol)

    e_latent_loss = jnp.sum(loss_parts) / jnp.float32(n * _EMB_DIM)
    loss = 0.25 * e_latent_loss
    return (jnp.swapaxes(out_t, 1, 2), loss.reshape(()))


# speedup vs baseline: 14.9148x; 1.1360x over previous
import jax
import jax.numpy as jnp
import numpy as np
from jax.experimental import pallas as pl
from jax.experimental.pallas import tpu as pltpu

_NUM_EMB = 1024
_EMB_DIM = 64
_SEQ = 1024


_SLABS = 4


def _vq_slab(xt, wt, icol, w_norm):
    i_norm = jnp.sum(xt * xt, axis=0, keepdims=True)

    neg_mm = jax.lax.dot_general(
        wt * -2.0, xt, (((0,), (0,)), ((), ())),
        preferred_element_type=jnp.float32)

    _C = 256
    mn = None
    idx = None
    for c in range(_NUM_EMB // _C):
        wn_c = jax.lax.slice(w_norm, (c * _C, 0), ((c + 1) * _C, 1))
        nm_c = jax.lax.slice(neg_mm, (c * _C, 0), ((c + 1) * _C, _SEQ))
        d_c = (wn_c + i_norm) + nm_c
        cmn = jnp.min(d_c, axis=0, keepdims=True)
        icol_c = jax.lax.slice(icol, (c * _C, 0), ((c + 1) * _C, 1))
        cand_c = jnp.where(d_c == cmn, icol_c, jnp.float32(_NUM_EMB))
        cidx = jnp.min(cand_c, axis=0, keepdims=True)
        if mn is None:
            mn, idx = cmn, cidx
        else:
            upd = cmn < mn
            idx = jnp.where(upd, cidx, idx)
            mn = jnp.minimum(mn, cmn)

    one_hot = (icol == idx).astype(jnp.float32)
    q = jax.lax.dot_general(
        wt, one_hot, (((1,), (0,)), ((), ())),
        preferred_element_type=jnp.float32)

    return xt + (q - xt), jnp.sum((xt - q) ** 2)


def _vq_block_kernel(xt_ref, wt_ref, icol_ref, out_ref, loss_ref):
    wt = wt_ref[...]
    icol = icol_ref[...]

    ones_col = jnp.ones((_EMB_DIM, 1), dtype=jnp.float32)
    w_norm = jax.lax.dot_general(
        wt * wt, ones_col, (((0,), (0,)), ((), ())),
        preferred_element_type=jnp.float32)

    loss = jnp.zeros((), dtype=jnp.float32)
    for s in range(_SLABS):
        xt = xt_ref[s]
        out, part = _vq_slab(xt, wt, icol, w_norm)
        out_ref[s] = out
        loss = loss + part

    i = pl.program_id(0)
    prev = jnp.where(i == 0, jnp.zeros((1, 1, 1), jnp.float32),
                     loss_ref[...])
    acc = prev + loss.reshape(1, 1, 1)
    last = i == pl.num_programs(0) - 1
    loss_ref[...] = jnp.where(last, acc * jnp.float32(0.25 / (2.0 ** 20)),
                              acc)


@jax.jit
def kernel(input, W):
    shape = input.shape
    n = shape[0] * shape[1]
    grid = shape[0] // _SLABS

    xt = jnp.swapaxes(input, 1, 2)
    wt = W.T
    icol = jnp.asarray(np.arange(_NUM_EMB, dtype=np.float32)[:, None])

    out_t, loss_parts = pl.pallas_call(
        _vq_block_kernel,
        grid=(grid,),
        in_specs=[
            pl.BlockSpec((_SLABS, _EMB_DIM, _SEQ), lambda i: (i, 0, 0)),
            pl.BlockSpec((_EMB_DIM, _NUM_EMB), lambda i: (0, 0)),
            pl.BlockSpec((_NUM_EMB, 1), lambda i: (0, 0)),
        ],
        out_specs=[
            pl.BlockSpec((_SLABS, _EMB_DIM, _SEQ), lambda i: (i, 0, 0)),
            pl.BlockSpec((1, 1, 1), lambda i: (0, 0, 0)),
        ],
        out_shape=[
            jax.ShapeDtypeStruct((shape[0], _EMB_DIM, _SEQ), jnp.float32),
            jax.ShapeDtypeStruct((1, 1, 1), jnp.float32),
        ],
        compiler_params=pltpu.CompilerParams(
            dimension_semantics=("arbitrary",)),
    )(xt, wt, icol)

    return (jnp.swapaxes(out_t, 1, 2), loss_parts.reshape(()))
